# baseline (speedup 1.0000x reference)
import functools

import jax
import jax.numpy as jnp
from jax import lax
from jax.experimental import pallas as pl
from jax.experimental.pallas import tpu as pltpu
from jax.experimental.pallas import tpu_sc as plsc

_MAX_POS = 8192
_EMBEDDING_DIM = 16
_NUM_CORES = 2
_NUM_SUBCORES = 16
_NUM_WORKERS = _NUM_CORES * _NUM_SUBCORES
_ROWS_PER_WORKER = _MAX_POS // _NUM_WORKERS


@functools.partial(
    pl.kernel,
    out_type=jax.ShapeDtypeStruct((_MAX_POS, _EMBEDDING_DIM), jnp.float32),
    mesh=plsc.VectorSubcoreMesh(core_axis_name="c", subcore_axis_name="s"),
    scratch_types=[pltpu.VMEM((_ROWS_PER_WORKER, _EMBEDDING_DIM), jnp.float32)],
    compiler_params=pltpu.CompilerParams(use_tc_tiling_on_sc=True),
)
def _sc_row_copy(w_hbm, out_hbm, buf):
    wid = lax.axis_index("s") * _NUM_CORES + lax.axis_index("c")
    base = wid * _ROWS_PER_WORKER
    pltpu.sync_copy(w_hbm.at[pl.ds(base, _ROWS_PER_WORKER)], buf)
    pltpu.sync_copy(buf, out_hbm.at[pl.ds(base, _ROWS_PER_WORKER)])


def kernel(input, weights):
    del input
    return _sc_row_copy(weights)


# speedup vs baseline: 1.3395x; 1.3395x over previous
import functools

import jax
import jax.numpy as jnp
from jax import lax
from jax.experimental import pallas as pl
from jax.experimental.pallas import tpu as pltpu
from jax.experimental.pallas import tpu_sc as plsc

_MAX_POS = 8192
_EMBEDDING_DIM = 16
_NUM_CORES = 2
_NUM_SUBCORES = 16
_NUM_WORKERS = _NUM_CORES * _NUM_SUBCORES
_ROWS_PER_WORKER = _MAX_POS // _NUM_WORKERS


_TILE_ROWS = 8
_COLS_PER_WORKER = _MAX_POS // (_NUM_WORKERS // 2)


@functools.partial(
    pl.kernel,
    out_type=jax.ShapeDtypeStruct((_EMBEDDING_DIM, _MAX_POS), jnp.float32),
    mesh=plsc.VectorSubcoreMesh(core_axis_name="c", subcore_axis_name="s"),
    scratch_types=[pltpu.VMEM((_TILE_ROWS, _COLS_PER_WORKER), jnp.float32)],
    compiler_params=pltpu.CompilerParams(use_tc_tiling_on_sc=True),
)
def _sc_row_copy(wt_hbm, out_hbm, buf):
    wid = lax.axis_index("s") * _NUM_CORES + lax.axis_index("c")
    r = (wid % 2) * _TILE_ROWS
    c = (wid // 2) * _COLS_PER_WORKER
    pltpu.sync_copy(
        wt_hbm.at[pl.ds(r, _TILE_ROWS), pl.ds(c, _COLS_PER_WORKER)], buf
    )
    pltpu.sync_copy(
        buf, out_hbm.at[pl.ds(r, _TILE_ROWS), pl.ds(c, _COLS_PER_WORKER)]
    )


def kernel(input, weights):
    del input
    return _sc_row_copy(weights.T).T


# speedup vs baseline: 1.3466x; 1.0053x over previous
import functools

import jax
import jax.numpy as jnp
from jax import lax
from jax.experimental import pallas as pl
from jax.experimental.pallas import tpu as pltpu
from jax.experimental.pallas import tpu_sc as plsc

_MAX_POS = 8192
_EMBEDDING_DIM = 16
_NUM_CORES = 2
_NUM_SUBCORES = 16
_NUM_WORKERS = _NUM_CORES * _NUM_SUBCORES
_ROWS_PER_WORKER = _MAX_POS // _NUM_WORKERS


_TILE_ROWS = 8
_COLS_PER_WORKER = _MAX_POS // (_NUM_WORKERS // 2)


@functools.partial(
    pl.kernel,
    out_type=jax.ShapeDtypeStruct((_EMBEDDING_DIM, _MAX_POS), jnp.float32),
    mesh=plsc.VectorSubcoreMesh(core_axis_name="c", subcore_axis_name="s"),
    scratch_types=[pltpu.VMEM((_TILE_ROWS, _COLS_PER_WORKER), jnp.float32)],
    compiler_params=pltpu.CompilerParams(
        use_tc_tiling_on_sc=True, skip_device_barrier=True
    ),
)
def _sc_row_copy(wt_hbm, out_hbm, buf):
    wid = lax.axis_index("s") * _NUM_CORES + lax.axis_index("c")
    r = (wid % 2) * _TILE_ROWS
    c = (wid // 2) * _COLS_PER_WORKER
    pltpu.sync_copy(
        wt_hbm.at[pl.ds(r, _TILE_ROWS), pl.ds(c, _COLS_PER_WORKER)], buf
    )
    pltpu.sync_copy(
        buf, out_hbm.at[pl.ds(r, _TILE_ROWS), pl.ds(c, _COLS_PER_WORKER)]
    )


def kernel(input, weights):
    del input
    return _sc_row_copy(weights.T).T


# speedup vs baseline: 1.4243x; 1.0577x over previous
import functools

import jax
import jax.numpy as jnp
from jax import lax
from jax.experimental import pallas as pl
from jax.experimental.pallas import tpu as pltpu
from jax.experimental.pallas import tpu_sc as plsc

_MAX_POS = 8192
_EMBEDDING_DIM = 16
_NUM_CORES = 2
_NUM_SUBCORES = 16
_NUM_WORKERS = _NUM_CORES * _NUM_SUBCORES
_ROWS_PER_WORKER = _MAX_POS // _NUM_WORKERS


_TILE_ROWS = 8
_COLS_PER_WORKER = _MAX_POS // (_NUM_WORKERS // 2)


@functools.partial(
    pl.kernel,
    out_type=jax.ShapeDtypeStruct((_EMBEDDING_DIM, _MAX_POS), jnp.float32),
    mesh=plsc.ScalarSubcoreMesh(axis_name="c", num_cores=_NUM_CORES),
    scratch_types=[
        pltpu.MemorySpace.VMEM_SHARED((_TILE_ROWS, _MAX_POS), jnp.float32)
    ],
    compiler_params=pltpu.CompilerParams(
        use_tc_tiling_on_sc=True, skip_device_barrier=True
    ),
)
def _sc_row_copy(wt_hbm, out_hbm, spmem):
    r = lax.axis_index("c") * _TILE_ROWS
    pltpu.sync_copy(wt_hbm.at[pl.ds(r, _TILE_ROWS)], spmem)
    pltpu.sync_copy(spmem, out_hbm.at[pl.ds(r, _TILE_ROWS)])


def kernel(input, weights):
    del input
    return _sc_row_copy(weights.T).T


# speedup vs baseline: 1.4261x; 1.0013x over previous
import functools

import jax
import jax.numpy as jnp
from jax import lax
from jax.experimental import pallas as pl
from jax.experimental.pallas import tpu as pltpu
from jax.experimental.pallas import tpu_sc as plsc

_MAX_POS = 8192
_EMBEDDING_DIM = 16
_NUM_CORES = 2
_NUM_SUBCORES = 16
_NUM_WORKERS = _NUM_CORES * _NUM_SUBCORES
_ROWS_PER_WORKER = _MAX_POS // _NUM_WORKERS


_TILE_ROWS = 8
_COLS_PER_WORKER = _MAX_POS // (_NUM_WORKERS // 2)


@functools.partial(
    pl.kernel,
    out_type=jax.ShapeDtypeStruct((_EMBEDDING_DIM, _MAX_POS), jnp.float32),
    mesh=plsc.ScalarSubcoreMesh(axis_name="c", num_cores=_NUM_CORES),
    scratch_types=[
        pltpu.MemorySpace.VMEM_SHARED((_TILE_ROWS, _MAX_POS // 2), jnp.float32),
        pltpu.MemorySpace.VMEM_SHARED((_TILE_ROWS, _MAX_POS // 2), jnp.float32),
        pltpu.SemaphoreType.DMA,
        pltpu.SemaphoreType.DMA,
    ],
    compiler_params=pltpu.CompilerParams(
        use_tc_tiling_on_sc=True, skip_device_barrier=True
    ),
)
def _sc_row_copy(wt_hbm, out_hbm, buf0, buf1, sem0, sem1):
    r = lax.axis_index("c") * _TILE_ROWS
    half = _MAX_POS // 2
    rows = pl.ds(r, _TILE_ROWS)
    in0 = pltpu.async_copy(wt_hbm.at[rows, pl.ds(0, half)], buf0, sem0)
    in1 = pltpu.async_copy(wt_hbm.at[rows, pl.ds(half, half)], buf1, sem1)
    in0.wait()
    out0 = pltpu.async_copy(buf0, out_hbm.at[rows, pl.ds(0, half)], sem0)
    in1.wait()
    out1 = pltpu.async_copy(buf1, out_hbm.at[rows, pl.ds(half, half)], sem1)
    out0.wait()
    out1.wait()


def kernel(input, weights):
    del input
    return _sc_row_copy(weights.T).T


# speedup vs baseline: 1.4911x; 1.0455x over previous
import functools

import jax
import jax.numpy as jnp
from jax import lax
from jax.experimental import pallas as pl
from jax.experimental.pallas import tpu as pltpu
from jax.experimental.pallas import tpu_sc as plsc

_MAX_POS = 8192
_EMBEDDING_DIM = 16
_NUM_CORES = 2
_NUM_SUBCORES = 16
_NUM_WORKERS = _NUM_CORES * _NUM_SUBCORES
_ROWS_PER_WORKER = _MAX_POS // _NUM_WORKERS


_TILE_ROWS = 8
_COLS_PER_WORKER = _MAX_POS // (_NUM_WORKERS // 2)


@functools.partial(
    pl.kernel,
    out_type=jax.ShapeDtypeStruct((_EMBEDDING_DIM, _MAX_POS), jnp.float32),
    mesh=plsc.ScalarSubcoreMesh(axis_name="c", num_cores=1),
    scratch_types=[
        pltpu.MemorySpace.VMEM_SHARED((_TILE_ROWS, _MAX_POS), jnp.float32),
        pltpu.MemorySpace.VMEM_SHARED((_TILE_ROWS, _MAX_POS), jnp.float32),
        pltpu.SemaphoreType.DMA,
        pltpu.SemaphoreType.DMA,
    ],
    compiler_params=pltpu.CompilerParams(
        use_tc_tiling_on_sc=True, skip_device_barrier=True
    ),
)
def _sc_row_copy(wt_hbm, out_hbm, buf0, buf1, sem0, sem1):
    lo = pl.ds(0, _TILE_ROWS)
    hi = pl.ds(_TILE_ROWS, _TILE_ROWS)
    in0 = pltpu.async_copy(wt_hbm.at[lo], buf0, sem0)
    in1 = pltpu.async_copy(wt_hbm.at[hi], buf1, sem1)
    in0.wait()
    out0 = pltpu.async_copy(buf0, out_hbm.at[lo], sem0)
    in1.wait()
    out1 = pltpu.async_copy(buf1, out_hbm.at[hi], sem1)
    out0.wait()
    out1.wait()


def kernel(input, weights):
    del input
    return _sc_row_copy(weights.T).T


# speedup vs baseline: 1.4940x; 1.0020x over previous
import functools

import jax
import jax.numpy as jnp
from jax import lax
from jax.experimental import pallas as pl
from jax.experimental.pallas import tpu as pltpu
from jax.experimental.pallas import tpu_sc as plsc

_MAX_POS = 8192
_EMBEDDING_DIM = 16
_NUM_CORES = 2
_NUM_SUBCORES = 16
_NUM_WORKERS = _NUM_CORES * _NUM_SUBCORES
_ROWS_PER_WORKER = _MAX_POS // _NUM_WORKERS


_TILE_ROWS = 8
_COLS_PER_WORKER = _MAX_POS // (_NUM_WORKERS // 2)


@functools.partial(
    pl.kernel,
    out_type=jax.ShapeDtypeStruct((_EMBEDDING_DIM, _MAX_POS), jnp.float32),
    mesh=plsc.ScalarSubcoreMesh(axis_name="c", num_cores=1),
    scratch_types=[
        pltpu.MemorySpace.VMEM_SHARED((_TILE_ROWS, _MAX_POS), jnp.float32),
        pltpu.MemorySpace.VMEM_SHARED((_TILE_ROWS, _MAX_POS), jnp.float32),
        pltpu.SemaphoreType.DMA,
        pltpu.SemaphoreType.DMA,
        pltpu.SemaphoreType.DMA,
        pltpu.SemaphoreType.DMA,
    ],
    compiler_params=pltpu.CompilerParams(
        use_tc_tiling_on_sc=True, skip_device_barrier=True
    ),
)
def _sc_row_copy(wt_hbm, out_hbm, buf0, buf1, si0, si1, so0, so1):
    half = _MAX_POS // 2
    chunks = []
    for k in range(4):
        rows = pl.ds((k % 2) * _TILE_ROWS, _TILE_ROWS)
        cols = pl.ds((k // 2) * half, half)
        buf = (buf0, buf1)[k % 2]
        chunks.append((rows, cols, buf))
    sin = (si0, si1, so0, so1)
    ins = [
        pltpu.async_copy(wt_hbm.at[rows, cols], buf.at[:, cols], sin[k])
        for k, (rows, cols, buf) in enumerate(chunks)
    ]
    outs = []
    for k, (rows, cols, buf) in enumerate(chunks):
        ins[k].wait()
        outs.append(
            pltpu.async_copy(buf.at[:, cols], out_hbm.at[rows, cols], sin[k])
        )
    for o in outs:
        o.wait()


def kernel(input, weights):
    del input
    return _sc_row_copy(weights.T).T
